# Initial kernel scaffold; baseline (speedup 1.0000x reference)
import jax
import jax.numpy as jnp
from jax.experimental import pallas as pl


def kernel(input_ids, type_ids, input_table, type_table, ln_gamma, ln_beta):
    raise NotImplementedError("write your pallas kernel here")


# speedup vs baseline: 5.0933x; 5.0933x over previous
import functools

import jax
import jax.numpy as jnp
from jax import lax
from jax.experimental import pallas as pl
from jax.experimental.pallas import tpu as pltpu
from jax.experimental.pallas import tpu_sc as plsc

EPS = 1e-12

NC = 2
NS = 16
NW = NC * NS

IROW = 128
G2 = 2


def _sc_gather(table, idx2d, n_tokens, d):
    n_rows = idx2d.shape[0]
    rows_per_tile = n_rows // NW
    steps = rows_per_tile // G2
    chunk = G2 * IROW

    mesh = plsc.VectorSubcoreMesh(core_axis_name="c", subcore_axis_name="s")

    @functools.partial(
        pl.kernel,
        out_type=jax.ShapeDtypeStruct((n_tokens, d), jnp.float32),
        mesh=mesh,
        scratch_types=[
            pltpu.VMEM((G2, IROW), jnp.int32),
            pltpu.VMEM((chunk, d), jnp.float32),
            pltpu.SemaphoreType.DMA,
        ],
    )
    def k(table_hbm, idx_hbm, out_hbm, idx_v, rows_v, sem):
        wid = lax.axis_index("s") * NC + lax.axis_index("c")
        row0 = wid * rows_per_tile

        def body(i, _):
            r = row0 + i * G2
            pltpu.sync_copy(idx_hbm.at[pl.ds(r, G2)], idx_v)
            cps = [
                pltpu.async_copy(
                    table_hbm.at[idx_v.at[j]],
                    rows_v.at[pl.ds(j * IROW, IROW)],
                    sem,
                )
                for j in range(G2)
            ]
            for c in cps:
                c.wait()
            pltpu.sync_copy(rows_v, out_hbm.at[pl.ds(r * IROW, chunk)])
            return ()

        lax.fori_loop(0, steps, body, (), unroll=False)

    return k(table, idx2d)


def _tc_type_ln(y, tids3, tt_pad, gamma2d, beta2d, tb, d):
    n_tokens = y.shape[0]
    grid = n_tokens // tb

    def body(ids_ref, y_ref, tt_ref, g_ref, b_ref, o_ref):
        ids = ids_ref[0, 0, :]
        oh = (ids[:, None] == lax.broadcasted_iota(jnp.int32, (tb, d), 1))
        temb = jax.lax.dot(
            oh.astype(jnp.float32), tt_ref[...],
            precision=jax.lax.Precision.HIGHEST,
        )
        x = y_ref[...] + temb
        mean = jnp.mean(x, axis=-1, keepdims=True)
        c = x - mean
        var = jnp.mean(c * c, axis=-1, keepdims=True)
        r = jax.lax.rsqrt(var + EPS)
        o_ref[...] = (c * r) * g_ref[...] + b_ref[...]

    return pl.pallas_call(
        body,
        grid=(grid,),
        in_specs=[
            pl.BlockSpec((1, 1, tb), lambda i: (i, 0, 0)),
            pl.BlockSpec((tb, d), lambda i: (i, 0)),
            pl.BlockSpec((d, d), lambda i: (0, 0)),
            pl.BlockSpec((1, d), lambda i: (0, 0)),
            pl.BlockSpec((1, d), lambda i: (0, 0)),
        ],
        out_specs=pl.BlockSpec((tb, d), lambda i: (i, 0)),
        out_shape=jax.ShapeDtypeStruct((n_tokens, d), jnp.float32),
    )(tids3, y, tt_pad, gamma2d, beta2d)


def kernel(input_ids, type_ids, input_table, type_table, ln_gamma, ln_beta):
    b, s = input_ids.shape
    d = input_table.shape[1]
    n_tokens = b * s

    idx2d = input_ids.reshape(n_tokens // IROW, IROW)
    y = _sc_gather(input_table, idx2d, n_tokens, d)

    tv = type_table.shape[0]
    tt_pad = jnp.zeros((d, d), jnp.float32).at[:tv].set(type_table)
    tb = 1024
    tids3 = type_ids.reshape(n_tokens // tb, 1, tb)
    out = _tc_type_ln(
        y, tids3, tt_pad,
        ln_gamma.reshape(1, d), ln_beta.reshape(1, d), tb, d,
    )
    return out.reshape(b, s, d)


# speedup vs baseline: 7.7393x; 1.5195x over previous
import functools

import jax
import jax.numpy as jnp
from jax import lax
from jax.experimental import pallas as pl
from jax.experimental.pallas import tpu as pltpu
from jax.experimental.pallas import tpu_sc as plsc

EPS = 1e-12

NC = 2
NS = 16
NW = NC * NS

IROW = 128
G2 = 2
NBUF = 2


def _sc_gather(table, idx2d, n_tokens, d):
    n_rows = idx2d.shape[0]
    rows_per_tile = n_rows // NW
    chunk = G2 * IROW
    chunks = rows_per_tile // G2
    pairs = chunks // NBUF

    mesh = plsc.VectorSubcoreMesh(core_axis_name="c", subcore_axis_name="s")

    @functools.partial(
        pl.kernel,
        out_type=jax.ShapeDtypeStruct((n_tokens, d), table.dtype),
        mesh=mesh,
        scratch_types=[
            pltpu.VMEM((rows_per_tile, IROW), jnp.int32),
            pltpu.VMEM((NBUF, chunk, d), table.dtype),
            pltpu.SemaphoreType.DMA,
            pltpu.SemaphoreType.DMA,
            pltpu.SemaphoreType.DMA,
            pltpu.SemaphoreType.DMA,
        ],
    )
    def k(table_hbm, idx_hbm, out_hbm, idx_v, rows_v, g0, g1, s0, s1):
        wid = lax.axis_index("s") * NC + lax.axis_index("c")
        row0 = wid * rows_per_tile
        tok0 = row0 * IROW
        gsem = (g0, g1)
        ssem = (s0, s1)

        pltpu.sync_copy(idx_hbm.at[pl.ds(row0, rows_per_tile)], idx_v)

        def fire(c, b):
            for j in range(G2):
                pltpu.async_copy(
                    table_hbm.at[idx_v.at[c * G2 + j]],
                    rows_v.at[b].at[pl.ds(j * IROW, IROW)],
                    gsem[b],
                )

        def drain_gather(b):
            pltpu.make_async_copy(
                out_hbm.at[pl.ds(0, chunk)], rows_v.at[b], gsem[b]
            ).wait()

        def scatter(c, b):
            pltpu.async_copy(
                rows_v.at[b],
                out_hbm.at[pl.ds(tok0 + c * chunk, chunk)],
                ssem[b],
            )

        def drain_scatter(b):
            pltpu.make_async_copy(
                rows_v.at[b], out_hbm.at[pl.ds(0, chunk)], ssem[b]
            ).wait()

        fire(0, 0)

        def body(i, _):
            c0 = i * NBUF
            drain_gather(0)
            scatter(c0, 0)

            @pl.when(i > 0)
            def _():
                drain_scatter(1)

            fire(c0 + 1, 1)

            drain_gather(1)
            scatter(c0 + 1, 1)
            drain_scatter(0)

            @pl.when(i < pairs - 1)
            def _():
                fire(c0 + 2, 0)

            return ()

        lax.fori_loop(0, pairs, body, (), unroll=False)
        drain_scatter(1)

    return k(table, idx2d)


def _tc_type_ln(y, tids3, tt_pad, gamma2d, beta2d, tb, d):
    n_tokens = y.shape[0]
    grid = n_tokens // tb

    def body(ids_ref, y_ref, tt_ref, g_ref, b_ref, o_ref):
        ids = ids_ref[0, 0, :]
        oh = (ids[:, None] == lax.broadcasted_iota(jnp.int32, (tb, d), 1))
        temb = jax.lax.dot(
            oh.astype(jnp.float32), tt_ref[...],
            precision=jax.lax.Precision.DEFAULT,
        )
        x = y_ref[...].astype(jnp.float32) + temb
        mean = jnp.mean(x, axis=-1, keepdims=True)
        c = x - mean
        var = jnp.mean(c * c, axis=-1, keepdims=True)
        r = jax.lax.rsqrt(var + EPS)
        o_ref[...] = (c * r) * g_ref[...] + b_ref[...]

    return pl.pallas_call(
        body,
        grid=(grid,),
        in_specs=[
            pl.BlockSpec((1, 1, tb), lambda i: (i, 0, 0)),
            pl.BlockSpec((tb, d), lambda i: (i, 0)),
            pl.BlockSpec((d, d), lambda i: (0, 0)),
            pl.BlockSpec((1, d), lambda i: (0, 0)),
            pl.BlockSpec((1, d), lambda i: (0, 0)),
        ],
        out_specs=pl.BlockSpec((tb, d), lambda i: (i, 0)),
        out_shape=jax.ShapeDtypeStruct((n_tokens, d), jnp.float32),
    )(tids3, y, tt_pad, gamma2d, beta2d)


def kernel(input_ids, type_ids, input_table, type_table, ln_gamma, ln_beta):
    b, s = input_ids.shape
    d = input_table.shape[1]
    n_tokens = b * s

    idx2d = input_ids.reshape(n_tokens // IROW, IROW)
    y = _sc_gather(input_table, idx2d, n_tokens, d)

    tv = type_table.shape[0]
    tt_pad = jnp.zeros((d, d), jnp.float32).at[:tv].set(type_table)
    tb = 2048
    tids3 = type_ids.reshape(n_tokens // tb, 1, tb)
    out = _tc_type_ln(
        y, tids3, tt_pad,
        ln_gamma.reshape(1, d), ln_beta.reshape(1, d), tb, d,
    )
    return out.reshape(b, s, d)


# speedup vs baseline: 9.3436x; 1.2073x over previous
import functools

import jax
import jax.numpy as jnp
from jax import lax
from jax.experimental import pallas as pl
from jax.experimental.pallas import tpu as pltpu
from jax.experimental.pallas import tpu_sc as plsc

EPS = 1e-12

NC = 2
NS = 16
NW = NC * NS

IROW = 128
G2 = 2
NBUF = 2


def _sc_gather(table, idx2d, n_tokens, d):
    n_rows = idx2d.shape[0]
    rows_per_tile = n_rows // NW
    chunk = G2 * IROW
    chunks = rows_per_tile // G2
    pairs = chunks // NBUF

    mesh = plsc.VectorSubcoreMesh(core_axis_name="c", subcore_axis_name="s")

    @functools.partial(
        pl.kernel,
        out_type=jax.ShapeDtypeStruct((n_tokens, d), table.dtype),
        mesh=mesh,
        scratch_types=[
            pltpu.VMEM((rows_per_tile, IROW), jnp.int32),
            pltpu.VMEM((NBUF, chunk, d), table.dtype),
            pltpu.SemaphoreType.DMA,
            pltpu.SemaphoreType.DMA,
            pltpu.SemaphoreType.DMA,
            pltpu.SemaphoreType.DMA,
        ],
    )
    def k(table_hbm, idx_hbm, out_hbm, idx_v, rows_v, g0, g1, s0, s1):
        wid = lax.axis_index("s") * NC + lax.axis_index("c")
        row0 = wid * rows_per_tile
        tok0 = row0 * IROW
        gsem = (g0, g1)
        ssem = (s0, s1)

        pltpu.sync_copy(idx_hbm.at[pl.ds(row0, rows_per_tile)], idx_v)

        def fire(c, b):
            for j in range(G2):
                pltpu.async_copy(
                    table_hbm.at[idx_v.at[c * G2 + j]],
                    rows_v.at[b].at[pl.ds(j * IROW, IROW)],
                    gsem[b],
                )

        def drain_gather(b):
            pltpu.make_async_copy(
                out_hbm.at[pl.ds(0, chunk)], rows_v.at[b], gsem[b]
            ).wait()

        def scatter(c, b):
            pltpu.async_copy(
                rows_v.at[b],
                out_hbm.at[pl.ds(tok0 + c * chunk, chunk)],
                ssem[b],
            )

        def drain_scatter(b):
            pltpu.make_async_copy(
                rows_v.at[b], out_hbm.at[pl.ds(0, chunk)], ssem[b]
            ).wait()

        fire(0, 0)

        def body(i, _):
            c0 = i * NBUF
            drain_gather(0)
            scatter(c0, 0)

            @pl.when(i > 0)
            def _():
                drain_scatter(1)

            fire(c0 + 1, 1)

            drain_gather(1)
            scatter(c0 + 1, 1)
            drain_scatter(0)

            @pl.when(i < pairs - 1)
            def _():
                fire(c0 + 2, 0)

            return ()

        lax.fori_loop(0, pairs, body, (), unroll=False)
        drain_scatter(1)

    return k(table, idx2d)


def _tc_type_ln(y, tids3, tt_pad, gamma2d, beta2d, tb, d):
    n_tokens = y.shape[0]
    grid = n_tokens // tb

    def body(ids_ref, y_ref, tt_ref, g_ref, b_ref, o_ref):
        ids = ids_ref[0, 0, :]
        oh = (ids[:, None] == lax.broadcasted_iota(jnp.int32, (tb, d), 1))
        temb = jax.lax.dot(
            oh.astype(jnp.float32), tt_ref[...],
            precision=jax.lax.Precision.DEFAULT,
        )
        x = y_ref[...] + temb
        jn = jnp.full((d, d), 1.0 / d, jnp.float32)
        mean = jax.lax.dot(x, jn, precision=jax.lax.Precision.DEFAULT)
        c = x - mean
        var = jax.lax.dot(c * c, jn, precision=jax.lax.Precision.DEFAULT)
        r = jax.lax.rsqrt(var + EPS)
        o_ref[...] = (c * r) * g_ref[...] + b_ref[...]

    return pl.pallas_call(
        body,
        grid=(grid,),
        in_specs=[
            pl.BlockSpec((1, 1, tb), lambda i: (i, 0, 0)),
            pl.BlockSpec((tb, d), lambda i: (i, 0)),
            pl.BlockSpec((d, d), lambda i: (0, 0)),
            pl.BlockSpec((1, d), lambda i: (0, 0)),
            pl.BlockSpec((1, d), lambda i: (0, 0)),
        ],
        out_specs=pl.BlockSpec((tb, d), lambda i: (i, 0)),
        out_shape=jax.ShapeDtypeStruct((n_tokens, d), jnp.float32),
    )(tids3, y, tt_pad, gamma2d, beta2d)


def kernel(input_ids, type_ids, input_table, type_table, ln_gamma, ln_beta):
    b, s = input_ids.shape
    d = input_table.shape[1]
    n_tokens = b * s

    idx2d = input_ids.reshape(n_tokens // IROW, IROW)
    y = _sc_gather(input_table, idx2d, n_tokens, d)

    tv = type_table.shape[0]
    tt_pad = jnp.zeros((d, d), jnp.float32).at[:tv].set(type_table)
    tb = 4096
    tids3 = type_ids.reshape(n_tokens // tb, 1, tb)
    out = _tc_type_ln(
        y, tids3, tt_pad,
        ln_gamma.reshape(1, d), ln_beta.reshape(1, d), tb, d,
    )
    return out.reshape(b, s, d)


# speedup vs baseline: 10.3536x; 1.1081x over previous
import functools

import jax
import jax.numpy as jnp
from jax import lax
from jax.experimental import pallas as pl
from jax.experimental.pallas import tpu as pltpu
from jax.experimental.pallas import tpu_sc as plsc

EPS = 1e-12

NC = 2
NS = 16
NW = NC * NS

IROW = 128
G2 = 2
NBUF = 2


def _sc_gather(table, idx2d, n_tokens, d):
    n_rows = idx2d.shape[0]
    rows_per_tile = n_rows // NW
    chunk = G2 * IROW
    chunks = rows_per_tile // G2
    pairs = chunks // NBUF

    mesh = plsc.VectorSubcoreMesh(core_axis_name="c", subcore_axis_name="s")

    @functools.partial(
        pl.kernel,
        out_type=jax.ShapeDtypeStruct((n_tokens, d), table.dtype),
        mesh=mesh,
        scratch_types=[
            pltpu.VMEM((rows_per_tile, IROW), jnp.int32),
            pltpu.VMEM((NBUF, chunk, d), table.dtype),
            pltpu.SemaphoreType.DMA,
            pltpu.SemaphoreType.DMA,
            pltpu.SemaphoreType.DMA,
            pltpu.SemaphoreType.DMA,
        ],
    )
    def k(table_hbm, idx_hbm, out_hbm, idx_v, rows_v, g0, g1, s0, s1):
        wid = lax.axis_index("s") * NC + lax.axis_index("c")
        row0 = wid * rows_per_tile
        tok0 = row0 * IROW
        gsem = (g0, g1)
        ssem = (s0, s1)

        pltpu.sync_copy(idx_hbm.at[pl.ds(row0, rows_per_tile)], idx_v)

        def fire(c, b):
            for j in range(G2):
                pltpu.async_copy(
                    table_hbm.at[idx_v.at[c * G2 + j]],
                    rows_v.at[b].at[pl.ds(j * IROW, IROW)],
                    gsem[b],
                )

        def drain_gather(b):
            pltpu.make_async_copy(
                out_hbm.at[pl.ds(0, chunk)], rows_v.at[b], gsem[b]
            ).wait()

        def scatter(c, b):
            pltpu.async_copy(
                rows_v.at[b],
                out_hbm.at[pl.ds(tok0 + c * chunk, chunk)],
                ssem[b],
            )

        def drain_scatter(b):
            pltpu.make_async_copy(
                rows_v.at[b], out_hbm.at[pl.ds(0, chunk)], ssem[b]
            ).wait()

        fire(0, 0)

        def body(i, _):
            c0 = i * NBUF
            drain_gather(0)
            scatter(c0, 0)

            @pl.when(i > 0)
            def _():
                drain_scatter(1)

            fire(c0 + 1, 1)

            drain_gather(1)
            scatter(c0 + 1, 1)
            drain_scatter(0)

            @pl.when(i < pairs - 1)
            def _():
                fire(c0 + 2, 0)

            return ()

        lax.fori_loop(0, pairs, body, (), unroll=False)
        drain_scatter(1)

    return k(table, idx2d)


def _tc_type_ln_slab(buf, y_k, tids3_k, tt_pad, gamma2d, beta2d,
                     tb, d, n_tokens, block0):
    slab_blocks = y_k.shape[0] // tb

    def body(*refs):
        if buf is None:
            ids_ref, y_ref, tt_ref, g_ref, b_ref, o_ref = refs
        else:
            _, ids_ref, y_ref, tt_ref, g_ref, b_ref, o_ref = refs
        ids = ids_ref[0, 0, :]
        oh = (ids[:, None] == lax.broadcasted_iota(jnp.int32, (tb, d), 1))
        temb = jax.lax.dot(
            oh.astype(jnp.float32), tt_ref[...],
            precision=jax.lax.Precision.DEFAULT,
        )
        x = y_ref[...] + temb
        jn = jnp.full((d, d), 1.0 / d, jnp.float32)
        mean = jax.lax.dot(x, jn, precision=jax.lax.Precision.DEFAULT)
        c = x - mean
        var = jax.lax.dot(c * c, jn, precision=jax.lax.Precision.DEFAULT)
        r = jax.lax.rsqrt(var + EPS)
        o_ref[...] = (c * r) * g_ref[...] + b_ref[...]

    in_specs = [
        pl.BlockSpec((1, 1, tb), lambda i: (i, 0, 0)),
        pl.BlockSpec((tb, d), lambda i: (i, 0)),
        pl.BlockSpec((d, d), lambda i: (0, 0)),
        pl.BlockSpec((1, d), lambda i: (0, 0)),
        pl.BlockSpec((1, d), lambda i: (0, 0)),
    ]
    args = [tids3_k, y_k, tt_pad, gamma2d, beta2d]
    aliases = {}
    if buf is not None:
        in_specs = [pl.BlockSpec(memory_space=pl.ANY)] + in_specs
        args = [buf] + args
        aliases = {0: 0}

    return pl.pallas_call(
        body,
        grid=(slab_blocks,),
        in_specs=in_specs,
        out_specs=pl.BlockSpec((tb, d), lambda i: (block0 + i, 0)),
        out_shape=jax.ShapeDtypeStruct((n_tokens, d), jnp.float32),
        input_output_aliases=aliases,
    )(*args)


K_SLABS = 5


def kernel(input_ids, type_ids, input_table, type_table, ln_gamma, ln_beta):
    b, s = input_ids.shape
    d = input_table.shape[1]
    n_tokens = b * s
    tb = 4096

    slab = n_tokens // K_SLABS
    idx2d = input_ids.reshape(n_tokens // IROW, IROW)
    tids3 = type_ids.reshape(n_tokens // tb, 1, tb)

    tv = type_table.shape[0]
    tt_pad = jnp.zeros((d, d), jnp.float32).at[:tv].set(type_table)
    g2 = ln_gamma.reshape(1, d)
    b2 = ln_beta.reshape(1, d)

    srows = slab // IROW
    sblocks = slab // tb
    ys = [
        _sc_gather(input_table, lax.slice_in_dim(idx2d, k * srows, (k + 1) * srows),
                   slab, d)
        for k in range(K_SLABS)
    ]
    buf = None
    for k in range(K_SLABS):
        buf = _tc_type_ln_slab(
            buf, ys[k],
            lax.slice_in_dim(tids3, k * sblocks, (k + 1) * sblocks),
            tt_pad, g2, b2, tb, d, n_tokens, k * sblocks,
        )
    return buf.reshape(b, s, d)
